# baseline (speedup 1.0000x reference)
import functools

import jax
import jax.numpy as jnp
from jax import lax
from jax.experimental import pallas as pl
from jax.experimental.pallas import tpu as pltpu
from jax.experimental.pallas import tpu_sc as plsc

_D = 32
_L = 16
_NC = 2
_NS = 16
_NW = _NC * _NS

_V = 768


def _relayout_call(tt):
    d, n = tt.shape
    n_out = -(-n // 8) * 8
    nblk = -(-n_out // _V)
    trips = -(-nblk // _NW)
    assert trips % 2 == 1 and trips >= 5

    mesh = plsc.VectorSubcoreMesh(core_axis_name="c", subcore_axis_name="s")

    @functools.partial(
        pl.kernel,
        mesh=mesh,
        out_type=jax.ShapeDtypeStruct((n_out, _D), jnp.float32),
        compiler_params=pltpu.CompilerParams(
            use_tc_tiling_on_sc=False, needs_layout_passes=False),
        scratch_types=[
            pltpu.VMEM((2, _D, _V + 1), jnp.float32),
            pltpu.VMEM((2, _V, _D), jnp.float32),
            pltpu.SemaphoreType.DMA,
            pltpu.SemaphoreType.DMA,
            pltpu.SemaphoreType.DMA,
            pltpu.SemaphoreType.DMA,
        ],
    )
    def k(tt_hbm, tab_hbm, cols_v, trans_v, sem_i0, sem_i1, sem_o0, sem_o1):
        wid = lax.axis_index("s") * _NC + lax.axis_index("c")
        sem_i = (sem_i0, sem_i1)
        sem_o = (sem_o0, sem_o1)

        def start(t):
            return pl.multiple_of(
                jnp.minimum((wid + _NW * t) * _V, n_out - _V), 8)

        def in_cp(t, slot):
            return pltpu.make_async_copy(
                tt_hbm.at[:, pl.ds(start(t), _V)],
                cols_v.at[slot, :, pl.ds(0, _V)], sem_i[slot])

        def out_cp(t, slot):
            return pltpu.make_async_copy(
                trans_v.at[slot],
                tab_hbm.at[pl.ds(start(t), _V), :], sem_o[slot])

        lanes = lax.iota(jnp.int32, _L)

        def transpose(slot):
            cols = cols_v.at[slot]

            def tbody(g, carry):
                for bl in range(_L):
                    b = g * _L + bl
                    bvec = jnp.full((_L,), b, jnp.int32)
                    for half in range(_D // _L):
                        didx = lanes + half * _L
                        vals = plsc.load_gather(cols, [didx, bvec])
                        trans_v[slot, b, pl.ds(half * _L, _L)] = vals
                return carry

            lax.fori_loop(0, _V // _L, tbody, 0)

        def step(t, slot, *, warmup=False, prefetch=True):
            in_cp(0, slot).wait()
            if prefetch:
                in_cp(t + 1, 1 - slot).start()
            if not warmup:
                out_cp(0, slot).wait()
            transpose(slot)
            out_cp(t, slot).start()

        in_cp(0, 0).start()
        step(0, 0, warmup=True)
        step(1, 1, warmup=True)

        def body(i, carry):
            step(2 * i, 0)
            step(2 * i + 1, 1)
            return carry

        lax.fori_loop(1, (trips - 1) // 2, body, 0)

        step(trips - 1, 0, prefetch=False)
        out_cp(0, 0).wait()
        out_cp(0, 1).wait()

    return k(tt)


def _gather_call(ids_t, table, *, batch, hist):
    nb = batch // _NW
    assert hist % 2 == 0 and hist >= 6 and nb % _L == 0

    mesh = plsc.VectorSubcoreMesh(core_axis_name="c", subcore_axis_name="s")

    @functools.partial(
        pl.kernel,
        mesh=mesh,
        out_type=jax.ShapeDtypeStruct((hist, _D, batch), jnp.float32),
        compiler_params=pltpu.CompilerParams(
            use_tc_tiling_on_sc=False, needs_layout_passes=False),
        scratch_types=[
            pltpu.VMEM((2, nb), jnp.int32),
            pltpu.VMEM((2, nb, _D), jnp.float32),
            pltpu.VMEM((2, _D, nb + 1), jnp.float32),
            pltpu.SemaphoreType.DMA,
            pltpu.SemaphoreType.DMA,
            pltpu.SemaphoreType.DMA,
            pltpu.SemaphoreType.DMA,
            pltpu.SemaphoreType.DMA,
            pltpu.SemaphoreType.DMA,
        ],
    )
    def k(ids_hbm, table_hbm, out_hbm, idx_v, rows_v, trans_v,
          sem_i0, sem_i1, sem_g0, sem_g1, sem_o0, sem_o1):
        wid = lax.axis_index("s") * _NC + lax.axis_index("c")
        b0 = wid * nb
        sem_i = (sem_i0, sem_i1)
        sem_g = (sem_g0, sem_g1)
        sem_o = (sem_o0, sem_o1)

        def idx_cp(h, slot):
            return pltpu.make_async_copy(
                ids_hbm.at[h, pl.ds(b0, nb)], idx_v.at[slot], sem_i[slot])

        def gather_cp(slot):
            return pltpu.make_async_copy(
                table_hbm.at[idx_v.at[slot]], rows_v.at[slot], sem_g[slot])

        def out_cp(h, slot):
            return pltpu.make_async_copy(
                trans_v.at[slot, :, pl.ds(0, nb)],
                out_hbm.at[h, :, pl.ds(b0, nb)], sem_o[slot])

        def plusone(slot):
            for t in range(nb // _L):
                sl = pl.ds(t * _L, _L)
                idx_v[slot, sl] = idx_v[slot, sl] + 1

        lanes = lax.iota(jnp.int32, _L)

        def transpose(slot):
            trans = trans_v.at[slot]

            def tbody(g, carry):
                for bl in range(_L):
                    b = g * _L + bl
                    bvec = jnp.full((_L,), b, jnp.int32)
                    for half in range(_D // _L):
                        didx = lanes + half * _L
                        vals = rows_v[slot, b, pl.ds(half * _L, _L)]
                        plsc.store_scatter(trans, [didx, bvec], vals)
                return carry

            lax.fori_loop(0, nb // _L, tbody, 0)

        def step(h, slot, *, warmup=False, prefetch=True, has_next=True):
            if has_next:
                idx_cp(0, 1 - slot).wait()
                plusone(1 - slot)
            gather_cp(slot).wait()
            if prefetch:
                idx_cp(h + 2, slot).start()
            if has_next:
                gather_cp(1 - slot).start()
            if not warmup:
                out_cp(0, slot).wait()
            transpose(slot)
            out_cp(h, slot).start()

        idx_cp(0, 0).start()
        idx_cp(1, 1).start()
        idx_cp(0, 0).wait()
        plusone(0)
        gather_cp(0).start()

        step(0, 0, warmup=True)
        step(1, 1, warmup=True)

        def body(i, carry):
            step(2 * i, 0)
            step(2 * i + 1, 1)
            return carry

        lax.fori_loop(1, hist // 2 - 1, body, 0)

        step(hist - 2, 0, prefetch=False)
        step(hist - 1, 1, prefetch=False, has_next=False)
        out_cp(0, 0).wait()
        out_cp(0, 1).wait()

    return k(ids_t, table)


def kernel(ids, table):
    b, h = ids.shape
    tab = _relayout_call(table.T)
    ids_t = ids.T
    out_t = _gather_call(ids_t, tab, batch=b, hist=h)
    return jnp.transpose(out_t, (2, 0, 1))


# speedup vs baseline: 4.0377x; 4.0377x over previous
import functools

import jax
import jax.numpy as jnp
from jax import lax
from jax.experimental import pallas as pl
from jax.experimental.pallas import tpu as pltpu
from jax.experimental.pallas import tpu_sc as plsc

_D = 32
_L = 16
_NC = 2
_NS = 16
_NW = _NC * _NS


def _gather_call(ids_t, table, *, batch, hist):
    nb = batch // _NW
    assert hist % 2 == 0 and hist >= 6 and nb % _L == 0

    mesh = plsc.VectorSubcoreMesh(core_axis_name="c", subcore_axis_name="s")

    @functools.partial(
        pl.kernel,
        mesh=mesh,
        out_type=jax.ShapeDtypeStruct((hist, _D, batch), jnp.float32),
        compiler_params=pltpu.CompilerParams(
            use_tc_tiling_on_sc=False, needs_layout_passes=False),
        scratch_types=[
            pltpu.VMEM((2, nb), jnp.int32),
            pltpu.VMEM((2, nb, _D), jnp.float32),
            pltpu.VMEM((2, _D, nb + 1), jnp.float32),
            pltpu.SemaphoreType.DMA,
            pltpu.SemaphoreType.DMA,
            pltpu.SemaphoreType.DMA,
            pltpu.SemaphoreType.DMA,
            pltpu.SemaphoreType.DMA,
            pltpu.SemaphoreType.DMA,
        ],
    )
    def k(ids_hbm, table_hbm, out_hbm, idx_v, rows_v, trans_v,
          sem_i0, sem_i1, sem_g0, sem_g1, sem_o0, sem_o1):
        wid = lax.axis_index("s") * _NC + lax.axis_index("c")
        b0 = wid * nb
        sem_i = (sem_i0, sem_i1)
        sem_g = (sem_g0, sem_g1)
        sem_o = (sem_o0, sem_o1)

        def idx_cp(h, slot):
            return pltpu.make_async_copy(
                ids_hbm.at[h, pl.ds(b0, nb)], idx_v.at[slot], sem_i[slot])

        def gather_cp(slot):
            return pltpu.make_async_copy(
                table_hbm.at[idx_v.at[slot]], rows_v.at[slot], sem_g[slot])

        def out_cp(h, slot):
            return pltpu.make_async_copy(
                trans_v.at[slot, :, pl.ds(0, nb)],
                out_hbm.at[h, :, pl.ds(b0, nb)], sem_o[slot])

        def plusone(slot):
            for t in range(nb // _L):
                sl = pl.ds(t * _L, _L)
                idx_v[slot, sl] = idx_v[slot, sl] + 1

        lanes = lax.iota(jnp.int32, _L)

        def transpose(slot):
            trans = trans_v.at[slot]

            def tbody(g, carry):
                for bl in range(_L):
                    b = g * _L + bl
                    bvec = jnp.full((_L,), b, jnp.int32)
                    for half in range(_D // _L):
                        didx = lanes + half * _L
                        vals = rows_v[slot, b, pl.ds(half * _L, _L)]
                        plsc.store_scatter(trans, [didx, bvec], vals)
                return carry

            lax.fori_loop(0, nb // _L, tbody, 0)

        def step(h, slot, *, warmup=False, prefetch=True, has_next=True):
            if has_next:
                idx_cp(0, 1 - slot).wait()
                plusone(1 - slot)
            gather_cp(slot).wait()
            if prefetch:
                idx_cp(h + 2, slot).start()
            if has_next:
                gather_cp(1 - slot).start()
            if not warmup:
                out_cp(0, slot).wait()
            transpose(slot)
            out_cp(h, slot).start()

        idx_cp(0, 0).start()
        idx_cp(1, 1).start()
        idx_cp(0, 0).wait()
        plusone(0)
        gather_cp(0).start()

        step(0, 0, warmup=True)
        step(1, 1, warmup=True)

        def body(i, carry):
            step(2 * i, 0)
            step(2 * i + 1, 1)
            return carry

        lax.fori_loop(1, hist // 2 - 1, body, 0)

        step(hist - 2, 0, prefetch=False)
        step(hist - 1, 1, prefetch=False, has_next=False)
        out_cp(0, 0).wait()
        out_cp(0, 1).wait()

    return k(ids_t, table)


def kernel(ids, table):
    b, h = ids.shape
    ids_t = ids.T
    out_t = _gather_call(ids_t, table, batch=b, hist=h)
    return jnp.transpose(out_t, (2, 0, 1))
